# baseline (speedup 1.0000x reference)
import functools

import jax
import jax.numpy as jnp
from jax import lax
from jax.experimental import pallas as pl
from jax.experimental.pallas import tpu as pltpu
from jax.experimental.pallas import tpu_sc as plsc

N = 100000
D = 128
H = 256
O = 10
G = 256
GA = G + 8
DUMP = G

NC = 2
NS = 16
NW = NC * NS
RPW = 3200
CH = 128
NCHUNK = RPW // CH
L = 16
DG = D // L

_mesh = plsc.VectorSubcoreMesh(
    core_axis_name="c", subcore_axis_name="s", num_cores=NC, num_subcores=NS)


@functools.partial(
    pl.kernel,
    out_type=[
        jax.ShapeDtypeStruct((NW, GA, D), jnp.float32),
        jax.ShapeDtypeStruct((NW, GA, L), jnp.float32),
    ],
    mesh=_mesh,
    scratch_types=[
        pltpu.VMEM((CH * D,), jnp.float32),
        pltpu.VMEM((CH * D,), jnp.float32),
        pltpu.VMEM((RPW,), jnp.int32),
        pltpu.VMEM((GA, D), jnp.float32),
        pltpu.VMEM((GA, L), jnp.float32),
        pltpu.SemaphoreType.DMA,
        pltpu.SemaphoreType.DMA,
    ],
    compiler_params=pltpu.CompilerParams(
        use_tc_tiling_on_sc=False, needs_layout_passes=False),
)
def _seg_sc(x_hbm, batch_hbm, zsum_hbm, zcnt_hbm,
            sums_out, cnt_out, buf_a, buf_b, idx_v, acc_v, cnt_v,
            sem_a, sem_b):
  cid = lax.axis_index("c")
  sid = lax.axis_index("s")
  wid = sid * NC + cid
  base = wid * RPW

  def start_fetch(c, buf, sem):
    off = jnp.minimum(base + c * CH, N - CH)
    pltpu.async_copy(x_hbm.at[pl.ds(off * D, CH * D)], buf, sem)

  def wait_fetch(buf, sem):
    pltpu.make_async_copy(x_hbm.at[pl.ds(0, CH * D)], buf, sem).wait()

  start_fetch(0, buf_a, sem_a)
  start_fetch(1, buf_b, sem_b)
  pltpu.sync_copy(batch_hbm.at[wid], idx_v)
  pltpu.sync_copy(zsum_hbm, acc_v)
  pltpu.sync_copy(zcnt_hbm, cnt_v)

  lane = lax.broadcasted_iota(jnp.int32, (L,), 0)
  ones16 = jnp.ones((L,), jnp.float32)

  def accumulate(c, buf):
    def rows(g, carry):
      r0 = g * L
      idv = idx_v[pl.ds(c * CH + r0, L)]
      for u in range(L):
        s = idv[u]
        srow = jnp.broadcast_to(s, (L,))
        rbase = (r0 + u) * D
        for j in range(DG):
          v = buf[pl.ds(rbase + j * L, L)]
          plsc.addupdate_scatter(acc_v, [srow, lane + (j * L)], v)
        plsc.addupdate_scatter(cnt_v, [srow, lane], ones16)
      return carry

    lax.fori_loop(0, CH // L, rows, 0)

  def body(k, carry):
    c = 2 * k
    wait_fetch(buf_a, sem_a)
    accumulate(c, buf_a)

    @pl.when(c + 2 < NCHUNK)
    def _():
      start_fetch(c + 2, buf_a, sem_a)

    wait_fetch(buf_b, sem_b)
    accumulate(c + 1, buf_b)

    @pl.when(c + 3 < NCHUNK)
    def _():
      start_fetch(c + 3, buf_b, sem_b)

    return carry

  lax.fori_loop(0, NCHUNK // 2, body, 0)

  wait_fetch(buf_a, sem_a)
  accumulate(NCHUNK - 1, buf_a)

  pltpu.sync_copy(acc_v, sums_out.at[wid])
  pltpu.sync_copy(cnt_v, cnt_out.at[wid])


def _mlp_kernel(sums_ref, cnt_ref, w1_ref, b1_ref, w2_ref, b2_ref,
                w3_ref, b3_ref, out_ref):
  sums = jnp.sum(sums_ref[...], axis=0)[:G]
  cnt = jnp.sum(cnt_ref[...], axis=0)[:G, 0:1]
  pooled = sums / jnp.maximum(cnt, 1.0)
  h = jnp.maximum(
      jnp.dot(pooled, w1_ref[...], preferred_element_type=jnp.float32)
      + b1_ref[...], 0.0)
  h = jnp.maximum(
      jnp.dot(h, w2_ref[...], preferred_element_type=jnp.float32)
      + b2_ref[...], 0.0)
  out_ref[...] = (
      jnp.dot(h, w3_ref[...], preferred_element_type=jnp.float32)
      + b3_ref[...])


def _build_ids(batch):
  ids = jnp.full((NW * RPW,), DUMP, dtype=jnp.int32)
  b = batch.astype(jnp.int32)
  full_rows = (N // CH) * CH
  ids = ids.at[:full_rows].set(b[:full_rows])
  own = N - full_rows
  ids = ids.at[full_rows + CH - own:full_rows + CH].set(b[N - own:])
  return ids.reshape(NW, RPW)


@jax.jit
def kernel(x, batch, W1, b1, W2, b2, W3, b3):
  ids = _build_ids(batch)
  xf = x.reshape(N * D)
  zsum = jnp.zeros((GA, D), jnp.float32)
  zcnt = jnp.zeros((GA, L), jnp.float32)
  sums2, cnt2 = _seg_sc(xf, ids, zsum, zcnt)
  out = pl.pallas_call(
      _mlp_kernel,
      out_shape=jax.ShapeDtypeStruct((G, O), jnp.float32),
  )(sums2, cnt2, W1, b1.reshape(1, H), W2, b2.reshape(1, H), W3,
    b3.reshape(1, O))
  return out
